# Initial kernel scaffold; baseline (speedup 1.0000x reference)
import jax
import jax.numpy as jnp
from jax.experimental import pallas as pl


def kernel(c_q_w, c_q_b, c_kv_w, c_kv_b, c_proj_w, c_proj_b, x, data):
    raise NotImplementedError("write your pallas kernel here")


# speedup vs baseline: 1.5870x; 1.5870x over previous
import functools

import jax
import jax.numpy as jnp
from jax import lax
from jax.experimental import pallas as pl
from jax.experimental.pallas import tpu as pltpu


def _fused_xattn_kernel(x_ref, data_ref, wq_ref, bq_ref, wkv_ref, bkv_ref,
                        wp_ref, bp_ref, o_ref, kv_sc, *, heads, attn_ch, width):
    @pl.when(pl.program_id(1) == 0)
    def _():
        kv = jnp.dot(data_ref[0], wkv_ref[...],
                     preferred_element_type=jnp.float32)
        kv_sc[...] = (kv + bkv_ref[...]).astype(jnp.bfloat16)

    q = (jnp.dot(x_ref[0], wq_ref[...], preferred_element_type=jnp.float32)
         + bq_ref[...]).astype(jnp.bfloat16)

    outs = []
    for h in range(heads):
        hs = h * attn_ch
        q_h = q[:, hs:hs + attn_ch]
        k_h = kv_sc[:, hs:hs + attn_ch]
        v_h = kv_sc[:, width + hs:width + hs + attn_ch]
        s = lax.dot_general(q_h, k_h, (((1,), (1,)), ((), ())),
                            preferred_element_type=jnp.float32)
        m = jnp.max(s, axis=-1, keepdims=True)
        p = jnp.exp(s - m)
        l = jnp.sum(p, axis=-1, keepdims=True)
        o_h = jnp.dot(p.astype(jnp.bfloat16), v_h,
                      preferred_element_type=jnp.float32)
        outs.append(o_h / l)

    o = jnp.concatenate(outs, axis=-1).astype(jnp.bfloat16)
    o_ref[0] = (jnp.dot(o, wp_ref[...], preferred_element_type=jnp.float32)
                + bp_ref[...])


def _pick_tq(n_ctx, target=256):
    if n_ctx <= target:
        return n_ctx
    t = target
    while t >= 8:
        if n_ctx % t == 0:
            return t
        t -= 8
    return n_ctx


def kernel(c_q_w, c_q_b, c_kv_w, c_kv_b, c_proj_w, c_proj_b, x, data):
    bs, n_ctx, width = x.shape
    _, n_data, data_width = data.shape
    heads = 8
    attn_ch = width // heads
    scale2 = 1.0 / (attn_ch ** 0.5)

    idx = jnp.arange(width)
    h_idx = idx // attn_ch
    c_idx = idx % attn_ch
    k_cols = h_idx * (2 * attn_ch) + c_idx
    perm = jnp.concatenate([k_cols, k_cols + attn_ch])
    wkv = c_kv_w[:, perm].astype(jnp.bfloat16)
    bkv = c_kv_b[perm].reshape(1, 2 * width)

    wq = (c_q_w * scale2).astype(jnp.bfloat16)
    bq = (c_q_b * scale2).reshape(1, width)
    wp = c_proj_w.astype(jnp.bfloat16)
    bp = c_proj_b.reshape(1, width)
    xb = x.astype(jnp.bfloat16)
    db = data.astype(jnp.bfloat16)

    tq = _pick_tq(n_ctx)
    kern = functools.partial(_fused_xattn_kernel, heads=heads,
                             attn_ch=attn_ch, width=width)
    out = pl.pallas_call(
        kern,
        out_shape=jax.ShapeDtypeStruct((bs, n_ctx, width), jnp.float32),
        grid=(bs, n_ctx // tq),
        in_specs=[
            pl.BlockSpec((1, tq, width), lambda b, i: (b, i, 0)),
            pl.BlockSpec((1, n_data, data_width), lambda b, i: (b, 0, 0)),
            pl.BlockSpec((width, width), lambda b, i: (0, 0)),
            pl.BlockSpec((1, width), lambda b, i: (0, 0)),
            pl.BlockSpec((data_width, 2 * width), lambda b, i: (0, 0)),
            pl.BlockSpec((1, 2 * width), lambda b, i: (0, 0)),
            pl.BlockSpec((width, width), lambda b, i: (0, 0)),
            pl.BlockSpec((1, width), lambda b, i: (0, 0)),
        ],
        out_specs=pl.BlockSpec((1, tq, width), lambda b, i: (b, i, 0)),
        scratch_shapes=[pltpu.VMEM((n_data, 2 * width), jnp.bfloat16)],
        compiler_params=pltpu.CompilerParams(
            dimension_semantics=("parallel", "arbitrary")
        ),
    )(xb, db, wq, bq, wkv, bkv, wp, bp)
    return out


# speedup vs baseline: 2.3395x; 1.4741x over previous
import functools

import jax
import jax.numpy as jnp
from jax import lax
from jax.experimental import pallas as pl
from jax.experimental.pallas import tpu as pltpu


def _fused_xattn_kernel(x_ref, data_ref, wq_ref, bq_ref, wkv_ref, bkv_ref,
                        wp_ref, bp_ref, o_ref, kv_sc, *, heads, attn_ch, width):
    @pl.when(pl.program_id(1) == 0)
    def _():
        kv = jnp.dot(data_ref[0], wkv_ref[...],
                     preferred_element_type=jnp.float32)
        kv_sc[...] = (kv + bkv_ref[...]).astype(jnp.bfloat16)

    q = (jnp.dot(x_ref[0], wq_ref[...], preferred_element_type=jnp.float32)
         + bq_ref[...]).astype(jnp.bfloat16)

    group = max(1, 256 // attn_ch)
    outs = []
    for h in range(heads):
        hs = h * attn_ch
        q_h = q[:, hs:hs + attn_ch]
        k_h = kv_sc[:, hs:hs + attn_ch]
        g = h // group
        gs = g * group * attn_ch
        v_g = kv_sc[:, width + gs:width + gs + group * attn_ch]
        s = lax.dot_general(q_h, k_h, (((1,), (1,)), ((), ())),
                            preferred_element_type=jnp.float32)
        m = jnp.max(s, axis=-1, keepdims=True)
        p = jnp.exp2(s - m)
        l = jnp.sum(p, axis=-1, keepdims=True)
        o_w = jnp.dot(p.astype(jnp.bfloat16), v_g,
                      preferred_element_type=jnp.float32)
        ls = (h % group) * attn_ch
        outs.append(o_w[:, ls:ls + attn_ch] / l)

    o = jnp.concatenate(outs, axis=-1).astype(jnp.bfloat16)
    o_ref[0] = (jnp.dot(o, wp_ref[...], preferred_element_type=jnp.float32)
                + bp_ref[...])


def _pick_tq(n_ctx, target=256):
    if n_ctx <= target:
        return n_ctx
    t = target
    while t >= 8:
        if n_ctx % t == 0:
            return t
        t -= 8
    return n_ctx


def kernel(c_q_w, c_q_b, c_kv_w, c_kv_b, c_proj_w, c_proj_b, x, data):
    bs, n_ctx, width = x.shape
    _, n_data, data_width = data.shape
    heads = 8
    attn_ch = width // heads
    scale2 = 1.4426950408889634 / (attn_ch ** 0.5)

    idx = jnp.arange(width)
    h_idx = idx // attn_ch
    c_idx = idx % attn_ch
    k_cols = h_idx * (2 * attn_ch) + c_idx
    perm = jnp.concatenate([k_cols, k_cols + attn_ch])
    wkv = c_kv_w[:, perm].astype(jnp.bfloat16)
    bkv = c_kv_b[perm].reshape(1, 2 * width)

    wq = (c_q_w * scale2).astype(jnp.bfloat16)
    bq = (c_q_b * scale2).reshape(1, width)
    wp = c_proj_w.astype(jnp.bfloat16)
    bp = c_proj_b.reshape(1, width)
    xb = x.astype(jnp.bfloat16)
    db = data.astype(jnp.bfloat16)

    tq = _pick_tq(n_ctx)
    kern = functools.partial(_fused_xattn_kernel, heads=heads,
                             attn_ch=attn_ch, width=width)
    out = pl.pallas_call(
        kern,
        out_shape=jax.ShapeDtypeStruct((bs, n_ctx, width), jnp.float32),
        grid=(bs, n_ctx // tq),
        in_specs=[
            pl.BlockSpec((1, tq, width), lambda b, i: (b, i, 0)),
            pl.BlockSpec((1, n_data, data_width), lambda b, i: (b, 0, 0)),
            pl.BlockSpec((width, width), lambda b, i: (0, 0)),
            pl.BlockSpec((1, width), lambda b, i: (0, 0)),
            pl.BlockSpec((data_width, 2 * width), lambda b, i: (0, 0)),
            pl.BlockSpec((1, 2 * width), lambda b, i: (0, 0)),
            pl.BlockSpec((width, width), lambda b, i: (0, 0)),
            pl.BlockSpec((1, width), lambda b, i: (0, 0)),
        ],
        out_specs=pl.BlockSpec((1, tq, width), lambda b, i: (b, i, 0)),
        scratch_shapes=[pltpu.VMEM((n_data, 2 * width), jnp.bfloat16)],
        compiler_params=pltpu.CompilerParams(
            dimension_semantics=("parallel", "arbitrary")
        ),
    )(xb, db, wq, bq, wkv, bkv, wp, bp)
    return out


# speedup vs baseline: 2.7107x; 1.1587x over previous
import functools

import jax
import jax.numpy as jnp
from jax import lax
from jax.experimental import pallas as pl
from jax.experimental.pallas import tpu as pltpu


def _fused_xattn_kernel(x_ref, data_ref, wq_ref, bq_ref, wkv_ref, bkv_ref,
                        wp_ref, bp_ref, o_ref, kv_sc, *, heads, attn_ch, width):
    @pl.when(pl.program_id(1) == 0)
    def _():
        kv = jnp.dot(data_ref[0], wkv_ref[...],
                     preferred_element_type=jnp.float32)
        kv_sc[...] = (kv + bkv_ref[...]).astype(jnp.bfloat16)

    q = (jnp.dot(x_ref[0], wq_ref[...], preferred_element_type=jnp.float32)
         + bq_ref[...]).astype(jnp.bfloat16)

    group = max(1, 256 // attn_ch)
    outs = []
    for h in range(heads):
        hs = h * attn_ch
        q_h = q[:, hs:hs + attn_ch]
        k_h = kv_sc[:, hs:hs + attn_ch]
        g = h // group
        gs = g * group * attn_ch
        v_g = kv_sc[:, width + gs:width + gs + group * attn_ch]
        s = lax.dot_general(q_h, k_h, (((1,), (1,)), ((), ())),
                            preferred_element_type=jnp.float32)
        m = jnp.max(s, axis=-1, keepdims=True)
        p = jnp.exp2(s - m)
        l = jnp.sum(p, axis=-1, keepdims=True)
        o_w = jnp.dot(p.astype(jnp.bfloat16), v_g,
                      preferred_element_type=jnp.float32)
        ls = (h % group) * attn_ch
        outs.append(o_w[:, ls:ls + attn_ch] / l)

    o = jnp.concatenate(outs, axis=-1).astype(jnp.bfloat16)
    o_ref[0] = (jnp.dot(o, wp_ref[...], preferred_element_type=jnp.float32)
                + bp_ref[...])


def _pick_tq(n_ctx, target=256):
    if n_ctx <= target:
        return n_ctx
    t = target
    while t >= 8:
        if n_ctx % t == 0:
            return t
        t -= 8
    return n_ctx


def kernel(c_q_w, c_q_b, c_kv_w, c_kv_b, c_proj_w, c_proj_b, x, data):
    bs, n_ctx, width = x.shape
    _, n_data, data_width = data.shape
    heads = 8
    attn_ch = width // heads
    scale2 = 1.4426950408889634 / (attn_ch ** 0.5)

    idx = jnp.arange(width)
    h_idx = idx // attn_ch
    c_idx = idx % attn_ch
    k_cols = h_idx * (2 * attn_ch) + c_idx
    perm = jnp.concatenate([k_cols, k_cols + attn_ch])
    wkv = c_kv_w[:, perm].astype(jnp.bfloat16)
    bkv = c_kv_b[perm].reshape(1, 2 * width)

    wq = (c_q_w * scale2).astype(jnp.bfloat16)
    bq = (c_q_b * scale2).reshape(1, width)
    wp = c_proj_w.astype(jnp.bfloat16)
    bp = c_proj_b.reshape(1, width)
    xb = x.astype(jnp.bfloat16)
    db = data.astype(jnp.bfloat16)

    tq = _pick_tq(n_ctx, 512)
    kern = functools.partial(_fused_xattn_kernel, heads=heads,
                             attn_ch=attn_ch, width=width)
    out = pl.pallas_call(
        kern,
        out_shape=jax.ShapeDtypeStruct((bs, n_ctx, width), jnp.float32),
        grid=(bs, n_ctx // tq),
        in_specs=[
            pl.BlockSpec((1, tq, width), lambda b, i: (b, i, 0)),
            pl.BlockSpec((1, n_data, data_width), lambda b, i: (b, 0, 0)),
            pl.BlockSpec((width, width), lambda b, i: (0, 0)),
            pl.BlockSpec((1, width), lambda b, i: (0, 0)),
            pl.BlockSpec((data_width, 2 * width), lambda b, i: (0, 0)),
            pl.BlockSpec((1, 2 * width), lambda b, i: (0, 0)),
            pl.BlockSpec((width, width), lambda b, i: (0, 0)),
            pl.BlockSpec((1, width), lambda b, i: (0, 0)),
        ],
        out_specs=pl.BlockSpec((1, tq, width), lambda b, i: (b, i, 0)),
        scratch_shapes=[pltpu.VMEM((n_data, 2 * width), jnp.bfloat16)],
        compiler_params=pltpu.CompilerParams(
            dimension_semantics=("parallel", "arbitrary")
        ),
    )(xb, db, wq, bq, wkv, bkv, wp, bp)
    return out


# speedup vs baseline: 2.8216x; 1.0409x over previous
import functools

import jax
import jax.numpy as jnp
from jax import lax
from jax.experimental import pallas as pl
from jax.experimental.pallas import tpu as pltpu


def _fused_xattn_kernel(x_ref, data_ref, wq_ref, bq_ref, wkv_ref, bkv_ref,
                        wp_ref, bp_ref, o_ref, kv_sc, *, heads, attn_ch, width):
    @pl.when(pl.program_id(1) == 0)
    def _():
        kv = jnp.dot(data_ref[0], wkv_ref[...],
                     preferred_element_type=jnp.float32)
        kv_sc[...] = (kv + bkv_ref[...]).astype(jnp.bfloat16)

    q = (jnp.dot(x_ref[0], wq_ref[...], preferred_element_type=jnp.float32)
         + bq_ref[...]).astype(jnp.bfloat16)

    group = max(1, 256 // attn_ch)
    outs = []
    for h in range(heads):
        hs = h * attn_ch
        q_h = q[:, hs:hs + attn_ch]
        k_h = kv_sc[:, hs:hs + attn_ch]
        g = h // group
        gs = g * group * attn_ch
        v_g = kv_sc[:, width + gs:width + gs + group * attn_ch]
        s = lax.dot_general(q_h, k_h, (((1,), (1,)), ((), ())),
                            preferred_element_type=jnp.float32)
        m = jnp.max(s, axis=-1, keepdims=True)
        p = jnp.exp2(s - m)
        l = jnp.sum(p, axis=-1, keepdims=True)
        o_w = jnp.dot(p.astype(jnp.bfloat16), v_g,
                      preferred_element_type=jnp.float32)
        ls = (h % group) * attn_ch
        outs.append(o_w[:, ls:ls + attn_ch] / l)

    o = jnp.concatenate(outs, axis=-1).astype(jnp.bfloat16)
    o_ref[0] = (jnp.dot(o, wp_ref[...], preferred_element_type=jnp.float32)
                + bp_ref[...])


def _pick_tq(n_ctx, target=256):
    if n_ctx <= target:
        return n_ctx
    t = target
    while t >= 8:
        if n_ctx % t == 0:
            return t
        t -= 8
    return n_ctx


def kernel(c_q_w, c_q_b, c_kv_w, c_kv_b, c_proj_w, c_proj_b, x, data):
    bs, n_ctx, width = x.shape
    _, n_data, data_width = data.shape
    heads = 8
    attn_ch = width // heads
    scale2 = 1.4426950408889634 / (attn_ch ** 0.5)

    idx = jnp.arange(width)
    h_idx = idx // attn_ch
    c_idx = idx % attn_ch
    k_cols = h_idx * (2 * attn_ch) + c_idx
    perm = jnp.concatenate([k_cols, k_cols + attn_ch])
    wkv = c_kv_w[:, perm].astype(jnp.bfloat16)
    bkv = c_kv_b[perm].reshape(1, 2 * width)

    wq = (c_q_w * scale2).astype(jnp.bfloat16)
    bq = (c_q_b * scale2).reshape(1, width)
    wp = c_proj_w.astype(jnp.bfloat16)
    bp = c_proj_b.reshape(1, width)
    xb = x.astype(jnp.bfloat16)
    db = data.astype(jnp.bfloat16)

    tq = _pick_tq(n_ctx, 1024)
    kern = functools.partial(_fused_xattn_kernel, heads=heads,
                             attn_ch=attn_ch, width=width)
    out = pl.pallas_call(
        kern,
        out_shape=jax.ShapeDtypeStruct((bs, n_ctx, width), jnp.float32),
        grid=(bs, n_ctx // tq),
        in_specs=[
            pl.BlockSpec((1, tq, width), lambda b, i: (b, i, 0)),
            pl.BlockSpec((1, n_data, data_width), lambda b, i: (b, 0, 0)),
            pl.BlockSpec((width, width), lambda b, i: (0, 0)),
            pl.BlockSpec((1, width), lambda b, i: (0, 0)),
            pl.BlockSpec((data_width, 2 * width), lambda b, i: (0, 0)),
            pl.BlockSpec((1, 2 * width), lambda b, i: (0, 0)),
            pl.BlockSpec((width, width), lambda b, i: (0, 0)),
            pl.BlockSpec((1, width), lambda b, i: (0, 0)),
        ],
        out_specs=pl.BlockSpec((1, tq, width), lambda b, i: (b, i, 0)),
        scratch_shapes=[pltpu.VMEM((n_data, 2 * width), jnp.bfloat16)],
        compiler_params=pltpu.CompilerParams(
            dimension_semantics=("parallel", "arbitrary")
        ),
    )(xb, db, wq, bq, wkv, bkv, wp, bp)
    return out


# speedup vs baseline: 2.8544x; 1.0116x over previous
import functools

import jax
import jax.numpy as jnp
from jax import lax
from jax.experimental import pallas as pl
from jax.experimental.pallas import tpu as pltpu


def _fused_xattn_kernel(x_ref, data_ref, wq_ref, bq_ref, wkv_ref, bkv_ref,
                        wp_ref, bp_ref, o_ref, kv_sc, *, heads, attn_ch, width):
    @pl.when(pl.program_id(1) == 0)
    def _():
        kv = jnp.dot(data_ref[0], wkv_ref[...],
                     preferred_element_type=jnp.float32)
        kv_sc[...] = (kv + bkv_ref[...]).astype(jnp.bfloat16)

    q = (jnp.dot(x_ref[0], wq_ref[...], preferred_element_type=jnp.float32)
         + bq_ref[...]).astype(jnp.bfloat16)

    outs = []
    for h in range(heads):
        hs = h * attn_ch
        q_h = q[:, hs:hs + attn_ch]
        k_h = kv_sc[:, hs:hs + attn_ch]
        vs = width + (h // 2) * 4 * attn_ch
        v_g = kv_sc[:, vs:vs + 4 * attn_ch]
        s = lax.dot_general(q_h, k_h, (((1,), (1,)), ((), ())),
                            preferred_element_type=jnp.float32)
        m = jnp.max(s, axis=-1, keepdims=True)
        p = jnp.exp2(s - m)
        o_w = jnp.dot(p.astype(jnp.bfloat16), v_g,
                      preferred_element_type=jnp.float32)
        l_w = o_w[:, 2 * attn_ch:3 * attn_ch]
        ls = (h % 2) * attn_ch
        outs.append(o_w[:, ls:ls + attn_ch] / l_w)

    o = jnp.concatenate(outs, axis=-1).astype(jnp.bfloat16)
    o_ref[0] = (jnp.dot(o, wp_ref[...], preferred_element_type=jnp.float32)
                + bp_ref[...])


def _pick_tq(n_ctx, target=256):
    if n_ctx <= target:
        return n_ctx
    t = target
    while t >= 8:
        if n_ctx % t == 0:
            return t
        t -= 8
    return n_ctx


def kernel(c_q_w, c_q_b, c_kv_w, c_kv_b, c_proj_w, c_proj_b, x, data):
    bs, n_ctx, width = x.shape
    _, n_data, data_width = data.shape
    heads = 8
    attn_ch = width // heads
    scale2 = 1.4426950408889634 / (attn_ch ** 0.5)

    idx = jnp.arange(width)
    h_idx = idx // attn_ch
    c_idx = idx % attn_ch
    k_cols = h_idx * (2 * attn_ch) + c_idx
    wk = c_kv_w[:, k_cols]
    bk = c_kv_b[k_cols]
    wv = c_kv_w[:, k_cols + attn_ch]
    bv = c_kv_b[k_cols + attn_ch]
    zw = jnp.zeros((data_width, attn_ch), c_kv_w.dtype)
    w_parts, b_parts = [wk], [bk]
    for g in range(heads // 2):
        w_parts += [wv[:, g * 2 * attn_ch:(g + 1) * 2 * attn_ch], zw, zw]
        b_parts += [bv[g * 2 * attn_ch:(g + 1) * 2 * attn_ch],
                    jnp.ones((attn_ch,), c_kv_b.dtype),
                    jnp.zeros((attn_ch,), c_kv_b.dtype)]
    kv_n = width + (heads // 2) * 4 * attn_ch
    wkv = jnp.concatenate(w_parts, axis=1).astype(jnp.bfloat16)
    bkv = jnp.concatenate(b_parts).reshape(1, kv_n)

    wq = (c_q_w * scale2).astype(jnp.bfloat16)
    bq = (c_q_b * scale2).reshape(1, width)
    wp = c_proj_w.astype(jnp.bfloat16)
    bp = c_proj_b.reshape(1, width)
    xb = x.astype(jnp.bfloat16)
    db = data.astype(jnp.bfloat16)

    tq = _pick_tq(n_ctx, 1024)
    kern = functools.partial(_fused_xattn_kernel, heads=heads,
                             attn_ch=attn_ch, width=width)
    out = pl.pallas_call(
        kern,
        out_shape=jax.ShapeDtypeStruct((bs, n_ctx, width), jnp.float32),
        grid=(bs, n_ctx // tq),
        in_specs=[
            pl.BlockSpec((1, tq, width), lambda b, i: (b, i, 0)),
            pl.BlockSpec((1, n_data, data_width), lambda b, i: (b, 0, 0)),
            pl.BlockSpec((width, width), lambda b, i: (0, 0)),
            pl.BlockSpec((1, width), lambda b, i: (0, 0)),
            pl.BlockSpec((data_width, kv_n), lambda b, i: (0, 0)),
            pl.BlockSpec((1, kv_n), lambda b, i: (0, 0)),
            pl.BlockSpec((width, width), lambda b, i: (0, 0)),
            pl.BlockSpec((1, width), lambda b, i: (0, 0)),
        ],
        out_specs=pl.BlockSpec((1, tq, width), lambda b, i: (b, i, 0)),
        scratch_shapes=[pltpu.VMEM((n_data, kv_n), jnp.bfloat16)],
        compiler_params=pltpu.CompilerParams(
            dimension_semantics=("parallel", "arbitrary")
        ),
    )(xb, db, wq, bq, wkv, bkv, wp, bp)
    return out
